# baseline (speedup 1.0000x reference)
import functools

import jax
import jax.numpy as jnp
from jax import lax
from jax.experimental import pallas as pl
from jax.experimental.pallas import tpu as pltpu
from jax.experimental.pallas import tpu_sc as plsc

N = 100000
D = 128
H = 256
B = 1024
M = 4096

NC = 2
NS = 16
NW = NC * NS

CHUNK = 3128
BKN = 224
NFULL = 13
ACC_ROWS = B + M
RPS = ACC_ROWS // NS
NCH = D // 16


def _sc_body(nf, sm, smf, bid, mid, waw, baw, zrows, parts,
             acc, fbuf, smb, smfb, bidb, midb, wawb, bawb,
             stage_g, stage_m, idxg, idxm, snapg, snapm):
    core = lax.axis_index("c")
    sid = lax.axis_index("s")
    wid = core * NS + sid

    pltpu.sync_copy(zrows.at[pl.ds(sid * RPS * D, RPS * D)],
                    acc.at[pl.ds(sid * RPS * D, RPS * D)])
    plsc.subcore_barrier()

    base = wid * CHUNK
    last = wid == NW - 1
    tail_off = jnp.where(last, 2808, 2904)
    tail_lo = jnp.where(last, 104, 8)

    pltpu.sync_copy(waw, wawb)
    pltpu.sync_copy(baw, bawb)
    wawc = [wawb[pl.ds(c * 16, 16)] for c in range(NCH)]
    bnval = -bawb[pl.ds(0, 16)][0]
    iota = lax.iota(jnp.int32, 16)
    zv = jnp.zeros((16,), jnp.float32)

    for c in range(NCH):
        snapg[pl.ds(c * 16, 16)] = zv
        snapm[pl.ds(c * 16, 16)] = zv

    def flush(stage, idx, snap, row, pfx):
        rb = row * D
        for c in range(NCH):
            s = snap[pl.ds(c * 16, 16)]
            stage[pl.ds(c * 16, 16)] = pfx[c] - s
            snap[pl.ds(c * 16, 16)] = pfx[c]
            idx[pl.ds(c * 16, 16)] = rb + c * 16 + iota
        pltpu.sync_copy(stage, acc.at[idx], add=True)

    def group_step(g, carry, lo):
        cur_b, cur_m, pg, pm = carry
        gb = g * 16
        bid16 = bidb[pl.ds(gb, 16)]
        mid16 = midb[pl.ds(gb, 16)]
        validm = (gb + iota) >= lo
        smv = jnp.where(validm, smb[pl.ds(gb, 16)], 0.0)
        smfv = jnp.where(validm & (mid16 > 0), smfb[pl.ds(gb, 16)], 0.0)
        for u in range(16):
            fc = [fbuf[pl.ds((gb + u) * D + c * 16, 16)] for c in range(NCH)]
            d = [fc[c] * wawc[c] for c in range(4)]
            for c in range(4, NCH):
                d[c - 4] = d[c - 4] + fc[c] * wawc[c]
            dv = (d[0] + d[1]) + (d[2] + d[3])
            for s in (8, 4, 2, 1):
                dv = dv + dv.at[iota ^ s].get(mode="promise_in_bounds")
            wv = 1.0 / (1.0 + jnp.exp(bnval - dv))
            wgu = wv * smv[u]
            wsu = wv * smfv[u]
            bid_u = bid16[u]
            mid_u = mid16[u]
            chg_b = bid_u != cur_b
            chg_m = mid_u != cur_m

            @pl.when(chg_b)
            def _(cb=cur_b, p=pg):
                flush(stage_g, idxg, snapg, cb, p)

            @pl.when(chg_m)
            def _(cm=cur_m, p=pm):
                flush(stage_m, idxm, snapm, B - 1 + cm, p)

            cur_b = jnp.where(chg_b, bid_u, cur_b)
            cur_m = jnp.where(chg_m, mid_u, cur_m)
            pg = [a + f * wgu for a, f in zip(pg, fc)]
            pm = [a + f * wsu for a, f in zip(pm, fc)]
        return cur_b, cur_m, pg, pm

    def block_step(b, carry):
        boff = lax.select(b == NFULL, tail_off, b * BKN)
        lo = lax.select(b == NFULL, tail_lo, 0)
        off = base + boff
        pltpu.sync_copy(nf.at[pl.ds(off * D, BKN * D)], fbuf)
        pltpu.sync_copy(sm.at[pl.ds(off, BKN)], smb.at[pl.ds(0, BKN)])
        pltpu.sync_copy(smf.at[pl.ds(off, BKN)], smfb.at[pl.ds(0, BKN)])
        pltpu.sync_copy(bid.at[pl.ds(off, BKN)], bidb.at[pl.ds(0, BKN)])
        pltpu.sync_copy(mid.at[pl.ds(off, BKN)], midb.at[pl.ds(0, BKN)])
        return lax.fori_loop(
            0, BKN // 16, lambda g, c: group_step(g, c, lo), carry)

    carry0 = (jnp.int32(0), jnp.int32(0), [zv] * NCH, [zv] * NCH)
    cur_b, cur_m, pg, pm = lax.fori_loop(0, NFULL + 1, block_step, carry0)
    flush(stage_g, idxg, snapg, cur_b, pg)
    flush(stage_m, idxm, snapm, B - 1 + cur_m, pm)

    plsc.subcore_barrier()
    pltpu.sync_copy(acc.at[pl.ds(sid * RPS * D, RPS * D)],
                    parts.at[core, pl.ds(sid * RPS * D, RPS * D)])


def _sc_pool(nf_flat, sm, smf, bid, mid, waw, baw16, zrows):
    return pl.kernel(
        _sc_body,
        out_type=jax.ShapeDtypeStruct((NC, ACC_ROWS * D), jnp.float32),
        mesh=plsc.VectorSubcoreMesh(core_axis_name="c", subcore_axis_name="s"),
        scratch_types=[
            pltpu.VMEM_SHARED((ACC_ROWS * D,), jnp.float32),
            pltpu.VMEM((BKN * D,), jnp.float32),
            pltpu.VMEM((BKN + 16,), jnp.float32),
            pltpu.VMEM((BKN + 16,), jnp.float32),
            pltpu.VMEM((BKN + 16,), jnp.int32),
            pltpu.VMEM((BKN + 16,), jnp.int32),
            pltpu.VMEM((D,), jnp.float32),
            pltpu.VMEM((16,), jnp.float32),
            pltpu.VMEM((D,), jnp.float32),
            pltpu.VMEM((D,), jnp.float32),
            pltpu.VMEM((D,), jnp.int32),
            pltpu.VMEM((D,), jnp.int32),
            pltpu.VMEM((D,), jnp.float32),
            pltpu.VMEM((D,), jnp.float32),
        ],
    )(nf_flat, sm, smf, bid, mid, waw, baw16, zrows)


def _mlp_body(p_ref, wf_ref, bf_ref, w1_ref, b1_ref, w2_ref, b2_ref,
              x_ref, o_ref):
    x = p_ref[0] + p_ref[1]
    x_ref[...] = x
    h0 = jnp.dot(x, wf_ref[...], preferred_element_type=jnp.float32) + bf_ref[...]
    h1 = jnp.maximum(
        jnp.dot(h0, w1_ref[...], preferred_element_type=jnp.float32) + b1_ref[...],
        0.0)
    o_ref[...] = jnp.dot(h1, w2_ref[...], preferred_element_type=jnp.float32) + b2_ref[...]


def kernel(node_feats, smask, smask_full, batch_ids, motif_ids,
           W_aw, b_aw, W_feat, b_feat, W1, b1, W2, b2):
    nf_flat = node_feats.reshape(-1)
    zrows = jnp.zeros((ACC_ROWS * D,), jnp.float32)
    parts = _sc_pool(nf_flat, smask, smask_full, batch_ids, motif_ids,
                     W_aw.reshape(D), jnp.pad(b_aw, (0, 15)), zrows)
    parts = parts.reshape(NC, ACC_ROWS, D)

    xsum, out = pl.pallas_call(
        _mlp_body,
        grid=(ACC_ROWS // 512,),
        in_specs=[
            pl.BlockSpec((NC, 512, D), lambda i: (0, i, 0)),
            pl.BlockSpec((D, H), lambda i: (0, 0)),
            pl.BlockSpec((1, H), lambda i: (0, 0)),
            pl.BlockSpec((H, H), lambda i: (0, 0)),
            pl.BlockSpec((1, H), lambda i: (0, 0)),
            pl.BlockSpec((H, H // 2), lambda i: (0, 0)),
            pl.BlockSpec((1, H // 2), lambda i: (0, 0)),
        ],
        out_specs=[
            pl.BlockSpec((512, D), lambda i: (i, 0)),
            pl.BlockSpec((512, H // 2), lambda i: (i, 0)),
        ],
        out_shape=[
            jax.ShapeDtypeStruct((ACC_ROWS, D), jnp.float32),
            jax.ShapeDtypeStruct((ACC_ROWS, H // 2), jnp.float32),
        ],
    )(parts, W_feat, b_feat.reshape(1, H), W1, b1.reshape(1, H),
      W2, b2.reshape(1, H // 2))

    return (xsum[:B], out[:B], out[B:])


# speedup vs baseline: 2.7319x; 2.7319x over previous
import jax
import jax.numpy as jnp
from jax import lax
from jax.experimental import pallas as pl
from jax.experimental.pallas import tpu as pltpu
from jax.experimental.pallas import tpu_sc as plsc

N = 100000
D = 128
H = 256
B = 1024
M = 4096

NC = 2
NS = 16
NW = NC * NS

BKN = 112
NFULL = 27
NBPW = 28
CHUNK = NBPW * BKN
NPAD = NW * CHUNK
TRASH = B + M
ACC_ROWS = 5248
RPS = ACC_ROWS // NS


PBN = 2048
PBLK = NPAD // PBN


def _gate_body(f_ref, sm_ref, smf_ref, mid_ref, waw_ref, baw_ref,
               wf_ref, ws_ref):
    f = f_ref[...]
    t = jnp.sum(f * waw_ref[...], axis=1) + baw_ref[0, 0]
    w = jax.nn.sigmoid(t)
    sm = sm_ref[0, 0, :]
    smf = smf_ref[0, 0, :]
    mid = mid_ref[0, 0, :]
    row = pl.program_id(0) * PBN + lax.broadcasted_iota(jnp.int32, (PBN, D), 0)
    valid = row < N
    wf_ref[...] = jnp.where(valid, f * (w * sm)[:, None], 0.0)
    ws_ref[...] = jnp.where(
        valid, f * (w * smf * (mid > 0).astype(jnp.float32))[:, None], 0.0)


def _gate(node_feats, smask, smask_full, motif_ids, W_aw, b_aw):
    pad = NPAD - N
    sm = jnp.pad(smask, (0, pad)).reshape(PBLK, 1, PBN)
    smf = jnp.pad(smask_full, (0, pad)).reshape(PBLK, 1, PBN)
    mid = jnp.pad(motif_ids, (0, pad)).reshape(PBLK, 1, PBN)
    return pl.pallas_call(
        _gate_body,
        grid=(PBLK,),
        in_specs=[
            pl.BlockSpec((PBN, D), lambda i: (i, 0)),
            pl.BlockSpec((1, 1, PBN), lambda i: (i, 0, 0)),
            pl.BlockSpec((1, 1, PBN), lambda i: (i, 0, 0)),
            pl.BlockSpec((1, 1, PBN), lambda i: (i, 0, 0)),
            pl.BlockSpec((1, D), lambda i: (0, 0)),
            pl.BlockSpec((1, 1), lambda i: (0, 0)),
        ],
        out_specs=[
            pl.BlockSpec((PBN, D), lambda i: (i, 0)),
            pl.BlockSpec((PBN, D), lambda i: (i, 0)),
        ],
        out_shape=[
            jax.ShapeDtypeStruct((NPAD, D), jnp.float32),
            jax.ShapeDtypeStruct((NPAD, D), jnp.float32),
        ],
    )(node_feats, sm, smf, mid, W_aw.reshape(1, D), b_aw.reshape(1, 1))


def _sc_body(wf_h, ws_h, bidx_h, midx_h, zrows, parts,
             acc, fwf0, fwf1, fws0, fws1, bib0, bib1, mib0, mib1,
             sin0, sin1, ssc0, ssc1):
    core = lax.axis_index("c")
    sid = lax.axis_index("s")
    wid = core * NS + sid

    fwf = (fwf0, fwf1)
    fws = (fws0, fws1)
    bib = (bib0, bib1)
    mib = (mib0, mib1)
    sin = (sin0, sin1)
    ssc = (ssc0, ssc1)

    pltpu.sync_copy(zrows.at[pl.ds(sid * RPS, RPS), :],
                    acc.at[pl.ds(sid * RPS, RPS), :])
    plsc.subcore_barrier()

    base = wid * CHUNK

    def off_of(bi):
        return pl.multiple_of(base + jnp.minimum(bi, NFULL) * BKN, 8)

    def issue_in(p, bi):
        off = off_of(bi)
        pltpu.async_copy(wf_h.at[pl.ds(off, BKN), :], fwf[p], sin[p])
        pltpu.async_copy(ws_h.at[pl.ds(off, BKN), :], fws[p], sin[p])
        pltpu.async_copy(bidx_h.at[pl.ds(off, BKN)], bib[p], sin[p])
        pltpu.async_copy(midx_h.at[pl.ds(off, BKN)], mib[p], sin[p])

    def wait_in(p, bi):
        off = off_of(bi)
        pltpu.make_async_copy(wf_h.at[pl.ds(off, BKN), :], fwf[p], sin[p]).wait()
        pltpu.make_async_copy(ws_h.at[pl.ds(off, BKN), :], fws[p], sin[p]).wait()
        pltpu.make_async_copy(bidx_h.at[pl.ds(off, BKN)], bib[p], sin[p]).wait()
        pltpu.make_async_copy(midx_h.at[pl.ds(off, BKN)], mib[p], sin[p]).wait()

    def issue_sc(p):
        pltpu.async_copy(fwf[p], acc.at[bib[p]], ssc[p], add=True)
        pltpu.async_copy(fws[p], acc.at[mib[p]], ssc[p], add=True)

    def wait_sc(p):
        pltpu.make_async_copy(fwf[p], acc.at[bib[p]], ssc[p]).wait()
        pltpu.make_async_copy(fws[p], acc.at[mib[p]], ssc[p]).wait()

    issue_in(0, jnp.int32(0))

    def body(i, carry):
        for p in (0, 1):
            bi = 2 * i + p
            wait_in(p, bi)

            @pl.when(bi >= 1)
            def _():
                wait_sc(1 - p)

            issue_in(1 - p, bi + 1)
            issue_sc(p)
        return carry

    lax.fori_loop(0, (NFULL + 1) // 2, body, jnp.int32(0))
    wait_in(0, jnp.int32(NFULL + 1))
    wait_sc(1)

    plsc.subcore_barrier()
    pltpu.sync_copy(acc.at[pl.ds(sid * RPS, RPS), :],
                    parts.at[core, pl.ds(sid * RPS, RPS), :])


def _sc_pool(wf, ws, bidx, midx, zrows):
    return pl.kernel(
        _sc_body,
        out_type=jax.ShapeDtypeStruct((NC, ACC_ROWS, D), jnp.float32),
        mesh=plsc.VectorSubcoreMesh(core_axis_name="c", subcore_axis_name="s"),
        scratch_types=[
            pltpu.VMEM_SHARED((ACC_ROWS, D), jnp.float32),
            pltpu.VMEM((BKN, D), jnp.float32),
            pltpu.VMEM((BKN, D), jnp.float32),
            pltpu.VMEM((BKN, D), jnp.float32),
            pltpu.VMEM((BKN, D), jnp.float32),
            pltpu.VMEM((BKN,), jnp.int32),
            pltpu.VMEM((BKN,), jnp.int32),
            pltpu.VMEM((BKN,), jnp.int32),
            pltpu.VMEM((BKN,), jnp.int32),
            pltpu.SemaphoreType.DMA,
            pltpu.SemaphoreType.DMA,
            pltpu.SemaphoreType.DMA,
            pltpu.SemaphoreType.DMA,
        ],
    )(wf, ws, bidx, midx, zrows)


def _mlp_body(p_ref, wf_ref, bf_ref, w1_ref, b1_ref, w2_ref, b2_ref,
              x_ref, o_ref):
    x = p_ref[0] + p_ref[1]
    x_ref[...] = x
    h0 = jnp.dot(x, wf_ref[...], preferred_element_type=jnp.float32) + bf_ref[...]
    h1 = jnp.maximum(
        jnp.dot(h0, w1_ref[...], preferred_element_type=jnp.float32) + b1_ref[...],
        0.0)
    o_ref[...] = jnp.dot(h1, w2_ref[...], preferred_element_type=jnp.float32) + b2_ref[...]


def kernel(node_feats, smask, smask_full, batch_ids, motif_ids,
           W_aw, b_aw, W_feat, b_feat, W1, b1, W2, b2):
    wf, ws = _gate(node_feats, smask, smask_full, motif_ids, W_aw, b_aw)
    pad = NPAD - N
    bidx = jnp.pad(batch_ids, (0, pad), constant_values=TRASH)
    midx = jnp.pad(motif_ids + (B - 1), (0, pad), constant_values=TRASH)
    zrows = jnp.zeros((ACC_ROWS, D), jnp.float32)
    parts = _sc_pool(wf, ws, bidx, midx, zrows)

    xsum, out = pl.pallas_call(
        _mlp_body,
        grid=((B + M) // 512,),
        in_specs=[
            pl.BlockSpec((NC, 512, D), lambda i: (0, i, 0)),
            pl.BlockSpec((D, H), lambda i: (0, 0)),
            pl.BlockSpec((1, H), lambda i: (0, 0)),
            pl.BlockSpec((H, H), lambda i: (0, 0)),
            pl.BlockSpec((1, H), lambda i: (0, 0)),
            pl.BlockSpec((H, H // 2), lambda i: (0, 0)),
            pl.BlockSpec((1, H // 2), lambda i: (0, 0)),
        ],
        out_specs=[
            pl.BlockSpec((512, D), lambda i: (i, 0)),
            pl.BlockSpec((512, H // 2), lambda i: (i, 0)),
        ],
        out_shape=[
            jax.ShapeDtypeStruct((B + M, D), jnp.float32),
            jax.ShapeDtypeStruct((B + M, H // 2), jnp.float32),
        ],
    )(parts, W_feat, b_feat.reshape(1, H), W1, b1.reshape(1, H),
      W2, b2.reshape(1, H // 2))

    return (xsum[:B], out[:B], out[B:])


# speedup vs baseline: 2.8542x; 1.0448x over previous
import jax
import jax.numpy as jnp
from jax import lax
from jax.experimental import pallas as pl
from jax.experimental.pallas import tpu as pltpu
from jax.experimental.pallas import tpu_sc as plsc

N = 100000
D = 128
H = 256
B = 1024
M = 4096

NC = 2
NS = 16
NW = NC * NS

BKN = 112
NFULL = 27
NBPW = 28
CHUNK = NBPW * BKN
NPAD = NW * CHUNK
TRASH = B + M
ACC_ROWS = 5248
RPS = ACC_ROWS // NS


PBN = 2048
PBLK = NPAD // PBN


def _gate_body(f_ref, sm_ref, smf_ref, mid_ref, waw_ref, baw_ref,
               wf_ref, ws_ref):
    f = f_ref[...]
    t = jnp.dot(f, waw_ref[...], preferred_element_type=jnp.float32)[:, 0:1]
    w = jax.nn.sigmoid(t + baw_ref[0, 0])
    sm = sm_ref[0, 0, :]
    smf = smf_ref[0, 0, :]
    mid = mid_ref[0, 0, :]
    row = pl.program_id(0) * PBN + lax.broadcasted_iota(jnp.int32, (PBN, D), 0)
    valid = row < N
    wf_ref[...] = jnp.where(valid, f * (w * sm[:, None]), 0.0)
    ws_ref[...] = jnp.where(
        valid,
        f * (w * (smf * (mid > 0).astype(jnp.float32))[:, None]), 0.0)


def _gate(node_feats, smask, smask_full, motif_ids, W_aw, b_aw):
    pad = NPAD - N
    sm = jnp.pad(smask, (0, pad)).reshape(PBLK, 1, PBN)
    smf = jnp.pad(smask_full, (0, pad)).reshape(PBLK, 1, PBN)
    mid = jnp.pad(motif_ids, (0, pad)).reshape(PBLK, 1, PBN)
    return pl.pallas_call(
        _gate_body,
        grid=(PBLK,),
        in_specs=[
            pl.BlockSpec((PBN, D), lambda i: (i, 0)),
            pl.BlockSpec((1, 1, PBN), lambda i: (i, 0, 0)),
            pl.BlockSpec((1, 1, PBN), lambda i: (i, 0, 0)),
            pl.BlockSpec((1, 1, PBN), lambda i: (i, 0, 0)),
            pl.BlockSpec((D, 8), lambda i: (0, 0)),
            pl.BlockSpec((1, 1), lambda i: (0, 0)),
        ],
        out_specs=[
            pl.BlockSpec((PBN, D), lambda i: (i, 0)),
            pl.BlockSpec((PBN, D), lambda i: (i, 0)),
        ],
        out_shape=[
            jax.ShapeDtypeStruct((NPAD, D), jnp.float32),
            jax.ShapeDtypeStruct((NPAD, D), jnp.float32),
        ],
    )(node_feats, sm, smf, mid, jnp.pad(W_aw, ((0, 0), (0, 7))),
      b_aw.reshape(1, 1))


def _sc_body(wf_h, ws_h, bidx_h, midx_h, zrows, parts,
             acc, fwf0, fwf1, fws0, fws1, bib0, bib1, mib0, mib1,
             sin0, sin1, ssc0, ssc1):
    core = lax.axis_index("c")
    sid = lax.axis_index("s")
    wid = core * NS + sid

    fwf = (fwf0, fwf1)
    fws = (fws0, fws1)
    bib = (bib0, bib1)
    mib = (mib0, mib1)
    sin = (sin0, sin1)
    ssc = (ssc0, ssc1)

    pltpu.sync_copy(zrows.at[pl.ds(sid * RPS, RPS), :],
                    acc.at[pl.ds(sid * RPS, RPS), :])
    plsc.subcore_barrier()

    base = wid * CHUNK

    def off_of(bi):
        return pl.multiple_of(base + jnp.minimum(bi, NFULL) * BKN, 8)

    def issue_in(p, bi):
        off = off_of(bi)
        pltpu.async_copy(wf_h.at[pl.ds(off, BKN), :], fwf[p], sin[p])
        pltpu.async_copy(ws_h.at[pl.ds(off, BKN), :], fws[p], sin[p])
        pltpu.async_copy(bidx_h.at[pl.ds(off, BKN)], bib[p], sin[p])
        pltpu.async_copy(midx_h.at[pl.ds(off, BKN)], mib[p], sin[p])

    def wait_in(p, bi):
        off = off_of(bi)
        pltpu.make_async_copy(wf_h.at[pl.ds(off, BKN), :], fwf[p], sin[p]).wait()
        pltpu.make_async_copy(ws_h.at[pl.ds(off, BKN), :], fws[p], sin[p]).wait()
        pltpu.make_async_copy(bidx_h.at[pl.ds(off, BKN)], bib[p], sin[p]).wait()
        pltpu.make_async_copy(midx_h.at[pl.ds(off, BKN)], mib[p], sin[p]).wait()

    def issue_sc(p):
        pltpu.async_copy(fwf[p], acc.at[bib[p]], ssc[p], add=True)
        pltpu.async_copy(fws[p], acc.at[mib[p]], ssc[p], add=True)

    def wait_sc(p):
        pltpu.make_async_copy(fwf[p], acc.at[bib[p]], ssc[p]).wait()
        pltpu.make_async_copy(fws[p], acc.at[mib[p]], ssc[p]).wait()

    issue_in(0, jnp.int32(0))

    def body(i, carry):
        for p in (0, 1):
            bi = 2 * i + p
            wait_in(p, bi)

            @pl.when(bi >= 1)
            def _():
                wait_sc(1 - p)

            issue_in(1 - p, bi + 1)
            issue_sc(p)
        return carry

    lax.fori_loop(0, (NFULL + 1) // 2, body, jnp.int32(0))
    wait_in(0, jnp.int32(NFULL + 1))
    wait_sc(1)

    plsc.subcore_barrier()
    pltpu.sync_copy(acc.at[pl.ds(sid * RPS, RPS), :],
                    parts.at[core, pl.ds(sid * RPS, RPS), :])


def _sc_pool(wf, ws, bidx, midx, zrows):
    return pl.kernel(
        _sc_body,
        out_type=jax.ShapeDtypeStruct((NC, ACC_ROWS, D), jnp.float32),
        mesh=plsc.VectorSubcoreMesh(core_axis_name="c", subcore_axis_name="s"),
        scratch_types=[
            pltpu.VMEM_SHARED((ACC_ROWS, D), jnp.float32),
            pltpu.VMEM((BKN, D), jnp.float32),
            pltpu.VMEM((BKN, D), jnp.float32),
            pltpu.VMEM((BKN, D), jnp.float32),
            pltpu.VMEM((BKN, D), jnp.float32),
            pltpu.VMEM((BKN,), jnp.int32),
            pltpu.VMEM((BKN,), jnp.int32),
            pltpu.VMEM((BKN,), jnp.int32),
            pltpu.VMEM((BKN,), jnp.int32),
            pltpu.SemaphoreType.DMA,
            pltpu.SemaphoreType.DMA,
            pltpu.SemaphoreType.DMA,
            pltpu.SemaphoreType.DMA,
        ],
    )(wf, ws, bidx, midx, zrows)


def _mlp_body(p_ref, wf_ref, bf_ref, w1_ref, b1_ref, w2_ref, b2_ref,
              x_ref, o_ref):
    x = p_ref[0] + p_ref[1]
    x_ref[...] = x
    h0 = jnp.dot(x, wf_ref[...], preferred_element_type=jnp.float32) + bf_ref[...]
    h1 = jnp.maximum(
        jnp.dot(h0, w1_ref[...], preferred_element_type=jnp.float32) + b1_ref[...],
        0.0)
    o_ref[...] = jnp.dot(h1, w2_ref[...], preferred_element_type=jnp.float32) + b2_ref[...]


def kernel(node_feats, smask, smask_full, batch_ids, motif_ids,
           W_aw, b_aw, W_feat, b_feat, W1, b1, W2, b2):
    wf, ws = _gate(node_feats, smask, smask_full, motif_ids, W_aw, b_aw)
    pad = NPAD - N
    bidx = jnp.pad(batch_ids, (0, pad), constant_values=TRASH)
    midx = jnp.pad(motif_ids + (B - 1), (0, pad), constant_values=TRASH)
    zrows = jnp.zeros((ACC_ROWS, D), jnp.float32)
    parts = _sc_pool(wf, ws, bidx, midx, zrows)

    xsum, out = pl.pallas_call(
        _mlp_body,
        grid=((B + M) // 512,),
        in_specs=[
            pl.BlockSpec((NC, 512, D), lambda i: (0, i, 0)),
            pl.BlockSpec((D, H), lambda i: (0, 0)),
            pl.BlockSpec((1, H), lambda i: (0, 0)),
            pl.BlockSpec((H, H), lambda i: (0, 0)),
            pl.BlockSpec((1, H), lambda i: (0, 0)),
            pl.BlockSpec((H, H // 2), lambda i: (0, 0)),
            pl.BlockSpec((1, H // 2), lambda i: (0, 0)),
        ],
        out_specs=[
            pl.BlockSpec((512, D), lambda i: (i, 0)),
            pl.BlockSpec((512, H // 2), lambda i: (i, 0)),
        ],
        out_shape=[
            jax.ShapeDtypeStruct((B + M, D), jnp.float32),
            jax.ShapeDtypeStruct((B + M, H // 2), jnp.float32),
        ],
    )(parts, W_feat, b_feat.reshape(1, H), W1, b1.reshape(1, H),
      W2, b2.reshape(1, H // 2))

    return (xsum[:B], out[:B], out[B:])


# speedup vs baseline: 3.0642x; 1.0736x over previous
import functools

import jax
import jax.numpy as jnp
from jax import lax
from jax.experimental import pallas as pl
from jax.experimental.pallas import tpu as pltpu
from jax.experimental.pallas import tpu_sc as plsc

N = 100000
D = 128
H = 256
B = 1024
M = 4096

NC = 2
NS = 16
NW = NC * NS

BKN = 112
NFULL = 13
NBPW = 14
CHUNK = NBPW * BKN
HALF = NW * CHUNK
NPAD = 2 * HALF
TRASH = B + M
ACC_ROWS = 5248
RPS = ACC_ROWS // NS


PBN = 1792
PBH = HALF // PBN
PBLK = NPAD // PBN


def _gate_body(kofs, f_ref, sm_ref, smf_ref, mid_ref, waw_ref, baw_ref,
               wf_ref, ws_ref):
    f = f_ref[...]
    t = jnp.dot(f, waw_ref[...], preferred_element_type=jnp.float32)[:, 0:1]
    w = jax.nn.sigmoid(t + baw_ref[0, 0])
    sm = sm_ref[0, 0, :]
    smf = smf_ref[0, 0, :]
    mid = mid_ref[0, 0, :]
    row = ((pl.program_id(0) + kofs) * PBN
           + lax.broadcasted_iota(jnp.int32, (PBN, D), 0))
    valid = row < N
    wf_ref[...] = jnp.where(valid, f * (w * sm[:, None]), 0.0)
    ws_ref[...] = jnp.where(
        valid,
        f * (w * (smf * (mid > 0).astype(jnp.float32))[:, None]), 0.0)


def _gate_half(k, node_feats, sm, smf, mid, waw8, baw):
    return pl.pallas_call(
        functools.partial(_gate_body, k * PBH),
        grid=(PBH,),
        in_specs=[
            pl.BlockSpec((PBN, D), lambda i: (i + k * PBH, 0)),
            pl.BlockSpec((1, 1, PBN), lambda i: (i + k * PBH, 0, 0)),
            pl.BlockSpec((1, 1, PBN), lambda i: (i + k * PBH, 0, 0)),
            pl.BlockSpec((1, 1, PBN), lambda i: (i + k * PBH, 0, 0)),
            pl.BlockSpec((D, 8), lambda i: (0, 0)),
            pl.BlockSpec((1, 1), lambda i: (0, 0)),
        ],
        out_specs=[
            pl.BlockSpec((PBN, D), lambda i: (i, 0)),
            pl.BlockSpec((PBN, D), lambda i: (i, 0)),
        ],
        out_shape=[
            jax.ShapeDtypeStruct((HALF, D), jnp.float32),
            jax.ShapeDtypeStruct((HALF, D), jnp.float32),
        ],
    )(node_feats, sm, smf, mid, waw8, baw)


def _sc_body(k, wf_h, ws_h, bidx_h, midx_h, zrows, parts,
             acc, fwf0, fwf1, fws0, fws1, bib0, bib1, mib0, mib1,
             sin0, sin1, ssc0, ssc1):
    core = lax.axis_index("c")
    sid = lax.axis_index("s")
    wid = core * NS + sid

    fwf = (fwf0, fwf1)
    fws = (fws0, fws1)
    bib = (bib0, bib1)
    mib = (mib0, mib1)
    sin = (sin0, sin1)
    ssc = (ssc0, ssc1)

    pltpu.sync_copy(zrows.at[pl.ds(sid * RPS, RPS), :],
                    acc.at[pl.ds(sid * RPS, RPS), :])
    plsc.subcore_barrier()

    base = wid * CHUNK

    def off_of(bi):
        return pl.multiple_of(base + jnp.minimum(bi, NFULL) * BKN, 8)

    def issue_in(p, bi):
        off = off_of(bi)
        goff = pl.multiple_of(k * HALF + off, 8)
        pltpu.async_copy(wf_h.at[pl.ds(off, BKN), :], fwf[p], sin[p])
        pltpu.async_copy(ws_h.at[pl.ds(off, BKN), :], fws[p], sin[p])
        pltpu.async_copy(bidx_h.at[pl.ds(goff, BKN)], bib[p], sin[p])
        pltpu.async_copy(midx_h.at[pl.ds(goff, BKN)], mib[p], sin[p])

    def wait_in(p, bi):
        off = off_of(bi)
        goff = pl.multiple_of(k * HALF + off, 8)
        pltpu.make_async_copy(wf_h.at[pl.ds(off, BKN), :], fwf[p], sin[p]).wait()
        pltpu.make_async_copy(ws_h.at[pl.ds(off, BKN), :], fws[p], sin[p]).wait()
        pltpu.make_async_copy(bidx_h.at[pl.ds(goff, BKN)], bib[p], sin[p]).wait()
        pltpu.make_async_copy(midx_h.at[pl.ds(goff, BKN)], mib[p], sin[p]).wait()

    def issue_sc(p):
        pltpu.async_copy(fwf[p], acc.at[bib[p]], ssc[p], add=True)
        pltpu.async_copy(fws[p], acc.at[mib[p]], ssc[p], add=True)

    def wait_sc(p):
        pltpu.make_async_copy(fwf[p], acc.at[bib[p]], ssc[p]).wait()
        pltpu.make_async_copy(fws[p], acc.at[mib[p]], ssc[p]).wait()

    issue_in(0, jnp.int32(0))

    def body(i, carry):
        for p in (0, 1):
            bi = 2 * i + p
            wait_in(p, bi)

            @pl.when(bi >= 1)
            def _():
                wait_sc(1 - p)

            issue_in(1 - p, bi + 1)
            issue_sc(p)
        return carry

    lax.fori_loop(0, (NFULL + 1) // 2, body, jnp.int32(0))
    wait_in(0, jnp.int32(NFULL + 1))
    wait_sc(1)

    plsc.subcore_barrier()
    pltpu.sync_copy(acc.at[pl.ds(sid * RPS, RPS), :],
                    parts.at[core, pl.ds(sid * RPS, RPS), :])


def _sc_pool(k, wf, ws, bidx, midx, zrows):
    return pl.kernel(
        functools.partial(_sc_body, k),
        out_type=jax.ShapeDtypeStruct((NC, ACC_ROWS, D), jnp.float32),
        mesh=plsc.VectorSubcoreMesh(core_axis_name="c", subcore_axis_name="s"),
        scratch_types=[
            pltpu.VMEM_SHARED((ACC_ROWS, D), jnp.float32),
            pltpu.VMEM((BKN, D), jnp.float32),
            pltpu.VMEM((BKN, D), jnp.float32),
            pltpu.VMEM((BKN, D), jnp.float32),
            pltpu.VMEM((BKN, D), jnp.float32),
            pltpu.VMEM((BKN,), jnp.int32),
            pltpu.VMEM((BKN,), jnp.int32),
            pltpu.VMEM((BKN,), jnp.int32),
            pltpu.VMEM((BKN,), jnp.int32),
            pltpu.SemaphoreType.DMA,
            pltpu.SemaphoreType.DMA,
            pltpu.SemaphoreType.DMA,
            pltpu.SemaphoreType.DMA,
        ],
    )(wf, ws, bidx, midx, zrows)


def _mlp_body(p_ref, q_ref, wf_ref, bf_ref, w1_ref, b1_ref, w2_ref, b2_ref,
              x_ref, o_ref):
    x = (p_ref[0] + p_ref[1]) + (q_ref[0] + q_ref[1])
    x_ref[...] = x
    h0 = jnp.dot(x, wf_ref[...], preferred_element_type=jnp.float32) + bf_ref[...]
    h1 = jnp.maximum(
        jnp.dot(h0, w1_ref[...], preferred_element_type=jnp.float32) + b1_ref[...],
        0.0)
    o_ref[...] = jnp.dot(h1, w2_ref[...], preferred_element_type=jnp.float32) + b2_ref[...]


def kernel(node_feats, smask, smask_full, batch_ids, motif_ids,
           W_aw, b_aw, W_feat, b_feat, W1, b1, W2, b2):
    pad = NPAD - N
    sm = jnp.pad(smask, (0, pad)).reshape(PBLK, 1, PBN)
    smf = jnp.pad(smask_full, (0, pad)).reshape(PBLK, 1, PBN)
    mid = jnp.pad(motif_ids, (0, pad)).reshape(PBLK, 1, PBN)
    waw8 = jnp.pad(W_aw, ((0, 0), (0, 7)))
    baw = b_aw.reshape(1, 1)
    bidx = jnp.pad(batch_ids, (0, pad), constant_values=TRASH)
    midx = jnp.pad(motif_ids + (B - 1), (0, pad), constant_values=TRASH)
    zrows = jnp.zeros((ACC_ROWS, D), jnp.float32)

    wf1, ws1 = _gate_half(0, node_feats, sm, smf, mid, waw8, baw)
    parts1 = _sc_pool(0, wf1, ws1, bidx, midx, zrows)
    wf2, ws2 = _gate_half(1, node_feats, sm, smf, mid, waw8, baw)
    parts2 = _sc_pool(1, wf2, ws2, bidx, midx, zrows)

    xsum, out = pl.pallas_call(
        _mlp_body,
        grid=((B + M) // 512,),
        in_specs=[
            pl.BlockSpec((NC, 512, D), lambda i: (0, i, 0)),
            pl.BlockSpec((NC, 512, D), lambda i: (0, i, 0)),
            pl.BlockSpec((D, H), lambda i: (0, 0)),
            pl.BlockSpec((1, H), lambda i: (0, 0)),
            pl.BlockSpec((H, H), lambda i: (0, 0)),
            pl.BlockSpec((1, H), lambda i: (0, 0)),
            pl.BlockSpec((H, H // 2), lambda i: (0, 0)),
            pl.BlockSpec((1, H // 2), lambda i: (0, 0)),
        ],
        out_specs=[
            pl.BlockSpec((512, D), lambda i: (i, 0)),
            pl.BlockSpec((512, H // 2), lambda i: (i, 0)),
        ],
        out_shape=[
            jax.ShapeDtypeStruct((B + M, D), jnp.float32),
            jax.ShapeDtypeStruct((B + M, H // 2), jnp.float32),
        ],
    )(parts1, parts2, W_feat, b_feat.reshape(1, H), W1, b1.reshape(1, H),
      W2, b2.reshape(1, H // 2))

    return (xsum[:B], out[:B], out[B:])
